# baseline (speedup 1.0000x reference)
import jax
import jax.numpy as jnp
from jax.experimental import pallas as pl
from jax.experimental.pallas import tpu as pltpu

_NCHUNKS = 8


def _dma_copy_kernel(in_ref, out_ref, sems):
    n = in_ref.shape[0]
    chunk = n // _NCHUNKS
    copies = []
    for i in range(_NCHUNKS):
        sl = pl.ds(i * chunk, chunk)
        copies.append(
            pltpu.make_async_copy(in_ref.at[sl], out_ref.at[sl], sems.at[i])
        )
    for c in copies:
        c.start()
    for c in copies:
        c.wait()


def kernel(x):
    b, c, h, w = x.shape
    xf = x.reshape(b * c, h, w)
    out = pl.pallas_call(
        _dma_copy_kernel,
        in_specs=[pl.BlockSpec(memory_space=pl.ANY)],
        out_specs=pl.BlockSpec(memory_space=pl.ANY),
        scratch_shapes=[pltpu.SemaphoreType.DMA((_NCHUNKS,))],
        out_shape=jax.ShapeDtypeStruct((b * c, h, w), x.dtype),
    )(xf)
    return out.reshape(x.shape)


# speedup vs baseline: 48.8839x; 48.8839x over previous
import jax
import jax.numpy as jnp
from jax.experimental import pallas as pl
from jax.experimental.pallas import tpu as pltpu

_ROWS = 4096


def _copy_kernel(in_ref, out_ref):
    out_ref[...] = in_ref[...]


def kernel(x):
    n = x.size // 512
    xf = x.reshape(n, 512)
    out = pl.pallas_call(
        _copy_kernel,
        grid=(n // _ROWS,),
        in_specs=[pl.BlockSpec((_ROWS, 512), lambda i: (i, 0))],
        out_specs=pl.BlockSpec((_ROWS, 512), lambda i: (i, 0)),
        out_shape=jax.ShapeDtypeStruct((n, 512), x.dtype),
        compiler_params=pltpu.CompilerParams(
            dimension_semantics=("parallel",),
        ),
    )(xf)
    return out.reshape(x.shape)


# speedup vs baseline: 49.1680x; 1.0058x over previous
import jax
import jax.numpy as jnp
from jax.experimental import pallas as pl
from jax.experimental.pallas import tpu as pltpu

_ROWS = 6144


def _copy_kernel(in_ref, out_ref):
    out_ref[...] = in_ref[...]


def kernel(x):
    n = x.size // 512
    xf = x.reshape(n, 512)
    out = pl.pallas_call(
        _copy_kernel,
        grid=(n // _ROWS,),
        in_specs=[pl.BlockSpec((_ROWS, 512), lambda i: (i, 0))],
        out_specs=pl.BlockSpec((_ROWS, 512), lambda i: (i, 0)),
        out_shape=jax.ShapeDtypeStruct((n, 512), x.dtype),
        compiler_params=pltpu.CompilerParams(
            dimension_semantics=("parallel",),
        ),
    )(xf)
    return out.reshape(x.shape)
